# Initial kernel scaffold; baseline (speedup 1.0000x reference)
import jax
import jax.numpy as jnp
from jax.experimental import pallas as pl


def kernel(esm_rep, seq, pssm, A, seq_embed, batch, params):
    raise NotImplementedError("write your pallas kernel here")


# speedup vs baseline: 2.0464x; 2.0464x over previous
import functools
import math

import jax
import jax.numpy as jnp
from jax.experimental import pallas as pl
from jax.experimental.pallas import tpu as pltpu

HEADS = 12


def _mm_body(x_ref, w_ref, o_ref, acc_ref, *, nk):
    k = pl.program_id(2)

    @pl.when(k == 0)
    def _():
        acc_ref[...] = jnp.zeros_like(acc_ref)

    acc_ref[...] += jnp.dot(x_ref[...], w_ref[...],
                            preferred_element_type=jnp.float32)

    @pl.when(k == nk - 1)
    def _():
        o_ref[...] = acc_ref[...]


def _ceil_to(v, m):
    return -(-v // m) * m


def _matmul(x, w):
    M, K = x.shape
    _, N = w.shape
    Mp = _ceil_to(M, 8) if M < 256 else _ceil_to(M, 256)
    bm = min(256, Mp)
    Kp = _ceil_to(K, 256)
    Np = _ceil_to(N, 256)
    bn = 256
    bk = 256
    xp = jnp.pad(x, ((0, Mp - M), (0, Kp - K)))
    wp = jnp.pad(w, ((0, Kp - K), (0, Np - N)))
    nk = Kp // bk
    out = pl.pallas_call(
        functools.partial(_mm_body, nk=nk),
        grid=(Mp // bm, Np // bn, nk),
        in_specs=[
            pl.BlockSpec((bm, bk), lambda i, j, k: (i, k)),
            pl.BlockSpec((bk, bn), lambda i, j, k: (k, j)),
        ],
        out_specs=pl.BlockSpec((bm, bn), lambda i, j, k: (i, j)),
        out_shape=jax.ShapeDtypeStruct((Mp, Np), jnp.float32),
        scratch_shapes=[pltpu.VMEM((bm, bn), jnp.float32)],
        compiler_params=pltpu.CompilerParams(
            dimension_semantics=("parallel", "parallel", "arbitrary")),
    )(xp, wp)
    return out[:M, :N]


def _gat(x, row2, col2, valid2, Wg, att_src, att_dst, out_ch):
    N, in_ch = x.shape
    Wg3 = Wg.reshape(in_ch, HEADS, out_ch)
    ws = jnp.einsum('ihc,hc->ih', Wg3, att_src)
    wd = jnp.einsum('ihc,hc->ih', Wg3, att_dst)
    a = _matmul(x, jnp.concatenate([ws, wd], axis=1))
    a_src = a[:, :HEADS]
    a_dst = a[:, HEADS:]

    alpha = jax.nn.leaky_relu(a_src[row2] + a_dst[col2], negative_slope=0.2)
    alpha = jnp.where(valid2[:, None], alpha, -1e9)
    amax = jax.ops.segment_max(alpha, col2, num_segments=N)
    ex = jnp.exp(alpha - amax[col2]) * valid2[:, None].astype(alpha.dtype)
    denom = jax.ops.segment_sum(ex, col2, num_segments=N)
    coef = ex / (denom[col2] + 1e-16)

    xg = x[row2]
    ms = [jax.ops.segment_sum(xg * coef[:, hd:hd + 1], col2, num_segments=N)
          for hd in range(HEADS)]
    m2 = jnp.concatenate(ms, axis=1)
    Wstack = Wg3.transpose(1, 0, 2).reshape(HEADS * in_ch, out_ch)
    return _matmul(m2, Wstack) * (1.0 / HEADS)


def _sag_pool(x, row, col, valid, Wrel, brel, Wroot):
    N = x.shape[0]
    vf = valid[:, None].astype(x.dtype)
    agg = jax.ops.segment_sum(x[row] * vf, col, num_segments=N)
    score = jnp.tanh(
        (_matmul(jnp.concatenate([agg, x], axis=1),
                 jnp.concatenate([Wrel, Wroot], axis=0)) + brel).reshape(-1))
    k = int(math.ceil(0.5 * N))
    _, perm = jax.lax.top_k(score, k)
    x_new = x[perm] * score[perm][:, None]
    new_idx = jnp.full((N,), -1, dtype=jnp.int32).at[perm].set(
        jnp.arange(k, dtype=jnp.int32))
    row_n = new_idx[row]
    col_n = new_idx[col]
    valid_n = valid & (row_n >= 0) & (col_n >= 0)
    row_n = jnp.where(valid_n, row_n, 0)
    col_n = jnp.where(valid_n, col_n, 0)
    return x_new, row_n, col_n, valid_n


def _with_loops(row, col, valid, N):
    loop = jnp.arange(N, dtype=row.dtype)
    row2 = jnp.concatenate([row, loop])
    col2 = jnp.concatenate([col, loop])
    valid2 = jnp.concatenate([valid, jnp.ones((N,), dtype=bool)])
    return row2, col2, valid2


def kernel(esm_rep, seq, pssm, A, seq_embed, batch, params):
    p = params
    N = seq.shape[2]
    x_seq = seq[0].T
    embed = jax.nn.relu(_matmul(x_seq, p['W_seq'].T) + p['b_seq'][None, :])

    row = A[0].astype(jnp.int32)
    col = A[1].astype(jnp.int32)
    valid = jnp.ones((row.shape[0],), dtype=bool)

    out = embed
    layer_cfg = [
        ('Wg1', 'as1', 'ad1', 'Wrel1', 'brel1', 'Wroot1', 512),
        ('Wg2', 'as2', 'ad2', 'Wrel2', 'brel2', 'Wroot2', 512),
        ('Wg3', 'as3', 'ad3', 'Wrel3', 'brel3', 'Wroot3', 1024),
        ('Wg4', 'as4', 'ad4', 'Wrel4', 'brel4', 'Wroot4', 1024),
    ]
    n_cur = N
    for (wg, asrc, adst, wrel, brel, wroot, oc) in layer_cfg:
        row2, col2, valid2 = _with_loops(row, col, valid, n_cur)
        out = _gat(out, row2, col2, valid2, p[wg], p[asrc], p[adst], oc)
        out, row, col, valid = _sag_pool(out, row, col, valid,
                                         p[wrel], p[brel], p[wroot])
        n_cur = out.shape[0]

    pooled = jnp.mean(out, axis=0, keepdims=True)
    feat = jnp.concatenate([pooled, seq_embed], axis=1)
    hdn = jax.nn.relu(_matmul(feat, p['Wc1']) + p['bc1'][None, :])
    return _matmul(hdn, p['Wc2']) + p['bc2'][None, :]
